# baseline (speedup 1.0000x reference)
import jax
import jax.numpy as jnp
from jax.experimental import pallas as pl

_BS, _T, _V, _F = 16, 32, 10, 2048
_N = _BS * _T
_VV = _V * _V
_LUT = 16
_G = 32


def _mm(a, b):
    return jax.lax.dot_general(a, b, (((1,), (0,)), ((), ())),
                               precision=jax.lax.Precision.HIGHEST,
                               preferred_element_type=jnp.float32)


def _iota2f(shape, dim):
    return jax.lax.broadcasted_iota(jnp.int32, shape, dim).astype(jnp.float32)


def _small_kernel(s_ref, mul_ref, bias_ref, means_ref, stds_ref,
                  emb3_ref, emb4_ref, ab_ref, d_ref):
    af = _iota2f((_VV, _VV), 0)
    bf = _iota2f((_VV, _VV), 1)
    bi = jnp.floor(bf * 0.1)
    bj = bf - 10.0 * bi
    ai = jnp.floor(af * 0.1)
    aj = af - 10.0 * ai

    rsel = _iota2f((_V, _VV), 0)
    cselj = _iota2f((_V, _VV), 1)
    cj = cselj - 10.0 * jnp.floor(cselj * 0.1)
    ciV = jnp.floor(cselj * 0.1)
    s = jnp.zeros((_N, _VV), jnp.float32)
    bias_f = jnp.zeros((1, _VV), jnp.float32)
    for i in range(_V):
        esel = ((rsel == cj) * (ciV == float(i))).astype(jnp.float32)
        s = s + _mm(s_ref[:, i, :], esel)
        bias_f = bias_f + _mm(bias_ref[i:i + 1, :], esel)

    rep = (rsel == cj).astype(jnp.float32)
    means_f = _mm(means_ref[...], rep)
    stds_f = _mm(stds_ref[...], rep)
    e3f = _mm(emb3_ref[0:2, :], rep)
    e4f = _mm(emb4_ref[...], rep)

    ra = _iota2f((_VV, _V), 0)
    rc = _iota2f((_VV, _V), 1)
    raj = ra - 10.0 * jnp.floor(ra * 0.1)
    rai = jnp.floor(ra * 0.1)
    rept = (rc == raj).astype(jnp.float32)

    q = _mm(rept, mul_ref[...])
    mul_t = _mm(q, rep)
    mker = jnp.where(ai == bi, mul_t, 0.0)

    perm = (af == bj * 10.0 + bi).astype(jnp.float32)
    smin = jnp.minimum(s, _mm(s, perm))

    ha = _iota2f((_VV, _LUT), 0)
    hv = _iota2f((_VV, _LUT), 1)
    hsel = ((ha - 10.0 * jnp.floor(ha * 0.1)) == hv).astype(jnp.float32)
    d_ref[...] = _mm(smin, hsel)

    h = _mm(smin, mker) + bias_f
    a = (2.0 * 3.14159) ** 0.5
    tmp = jnp.exp(-0.5 * ((h - means_f) / stds_f) ** 2) / (a * stds_f)
    ef = jnp.tanh(jax.nn.sigmoid(tmp))

    dist = smin
    sp = jnp.zeros((_N, _VV), jnp.float32)
    for k in range(_V):
        ck = ((af == bi * 10.0 + k).astype(jnp.float32)
              + (af == k * 10.0 + bj).astype(jnp.float32))
        temp = _mm(dist, ck)
        new = jnp.minimum(dist, temp)
        x = jnp.where(jnp.equal(new, dist), 0.0, 1.0)
        sp = sp + x * ef
        dist = new

    gsum = (ai == bi).astype(jnp.float32)

    cnt = _mm(dist, gsum)
    sp_enc = (_V - cnt) * e3f[0:1, :] + cnt * e3f[1:2, :]

    eidx = jnp.floor(sp)
    ed_enc = jnp.zeros((_N, _VV), jnp.float32)
    for dd in range(10):
        w = _mm((eidx == float(dd)).astype(jnp.float32), gsum)
        ed_enc = ed_enc + w * e4f[dd:dd + 1, :]

    ab = sp_enc + ed_enc

    for i in range(_V):
        esel_t = ((rc == raj) * (rai == float(i))).astype(jnp.float32)
        ab_ref[:, i, :] = _mm(ab, esel_t)


def _stream_kernel(x_ref, d_ref, ein_ref, eout_ref, o_ref):
    lut = ein_ref[...] + eout_ref[...]
    dv = d_ref[...].astype(jnp.int32)
    ids = jax.lax.broadcasted_iota(jnp.int32, (1, _LUT), 1)
    for v in range(_V):
        oh = (dv[:, v:v + 1] == ids).astype(jnp.float32)
        rows = _mm(oh, lut)
        o_ref[:, v, :] = x_ref[:, v, :] + rows


def kernel(end_output, S, emb_in, emb_out, emb3, emb4, mul, bias, means, stds):
    s3 = S.reshape(_N, _V, _V)

    ab3, d2 = pl.pallas_call(
        _small_kernel,
        out_shape=(jax.ShapeDtypeStruct((_N, _V, _V), jnp.float32),
                   jax.ShapeDtypeStruct((_N, _LUT), jnp.float32)),
    )(s3, mul, bias, means, stds, emb3, emb4)

    out3 = pl.pallas_call(
        _stream_kernel,
        grid=(_N // _G,),
        in_specs=[pl.BlockSpec((_G, _V, _F), lambda i: (i, 0, 0)),
                  pl.BlockSpec((_G, _LUT), lambda i: (i, 0)),
                  pl.BlockSpec((_LUT, _F), lambda i: (0, 0)),
                  pl.BlockSpec((_LUT, _F), lambda i: (0, 0))],
        out_specs=pl.BlockSpec((_G, _V, _F), lambda i: (i, 0, 0)),
        out_shape=jax.ShapeDtypeStruct((_N, _V, _F), jnp.float32),
    )(end_output.reshape(_N, _V, _F), d2, emb_in, emb_out)

    return (out3.reshape(_BS, _T, _V, _F),
            ab3.reshape(_BS, _T, _V, _V))


# speedup vs baseline: 4.9986x; 4.9986x over previous
import jax
import jax.numpy as jnp
from jax.experimental import pallas as pl
from jax.experimental.pallas import tpu as pltpu

_BS, _T, _V, _F = 16, 32, 10, 2048
_LUT = 16
_GB = 4


def _fused_kernel(s_ref, w_ref, x_ref, ein_ref, eout_ref,
                  ab_ref, o_ref, deg_ref):
    step = pl.program_id(0)

    @pl.when(step == 0)
    def _small():
        s = s_ref[...]
        dist = jnp.minimum(s, s_ref[...].transpose(1, 0, 2, 3))

        deg_ref[...] = jnp.sum(dist, axis=0)

        for k in range(_V):
            m = jnp.maximum(dist[:, k:k + 1, :, :], dist[k:k + 1, :, :, :])
            dist = dist * m

        cnt = jnp.sum(dist, axis=1)
        ab_ref[...] = ((_V - cnt)[:, None, :, :] * w_ref[0:1]
                       + cnt[:, None, :, :] * w_ref[1:2]
                       + float(_V) * w_ref[2:3])

    lut = ein_ref[...] + eout_ref[...]
    ci = jax.lax.broadcasted_iota(jnp.int32, (_LUT, 1), 0).astype(jnp.float32)
    for g in range(_GB):
        bb = step * _GB + g
        for v in range(_V):
            degv = deg_ref[v, pl.ds(bb, 1), :]
            oht = (ci == degv).astype(jnp.float32)
            rows = jax.lax.dot_general(
                oht, lut, (((0,), (0,)), ((), ())),
                preferred_element_type=jnp.float32)
            o_ref[g, v] = x_ref[g, v] + rows


def kernel(end_output, S, emb_in, emb_out, emb3, emb4, mul, bias, means, stds):
    s4 = jnp.transpose(S, (2, 3, 0, 1))
    x4 = jnp.transpose(end_output, (0, 2, 1, 3))

    wpack = jnp.concatenate(
        [emb3[0:2, :], emb4[0:1, :],
         jnp.zeros((_V - 3, _V), jnp.float32)], axis=0).reshape(_V, _V, 1, 1)

    ab4, o4 = pl.pallas_call(
        _fused_kernel,
        grid=(_BS // _GB,),
        in_specs=[pl.BlockSpec((_V, _V, _BS, _T), lambda i: (0, 0, 0, 0)),
                  pl.BlockSpec((_V, _V, 1, 1), lambda i: (0, 0, 0, 0)),
                  pl.BlockSpec((_GB, _V, _T, _F), lambda i: (i, 0, 0, 0)),
                  pl.BlockSpec((_LUT, _F), lambda i: (0, 0)),
                  pl.BlockSpec((_LUT, _F), lambda i: (0, 0))],
        out_specs=(pl.BlockSpec((_V, _V, _BS, _T), lambda i: (0, 0, 0, 0)),
                   pl.BlockSpec((_GB, _V, _T, _F), lambda i: (i, 0, 0, 0))),
        out_shape=(jax.ShapeDtypeStruct((_V, _V, _BS, _T), jnp.float32),
                   jax.ShapeDtypeStruct((_BS, _V, _T, _F), jnp.float32)),
        scratch_shapes=[pltpu.VMEM((_V, _BS, _T), jnp.float32)],
    )(s4, wpack, x4, emb_in, emb_out)

    return (jnp.transpose(o4, (0, 2, 1, 3)),
            jnp.transpose(ab4, (2, 3, 0, 1)))
